# Initial kernel scaffold; baseline (speedup 1.0000x reference)
import functools

import jax
import jax.numpy as jnp
from jax import lax
from jax.experimental import pallas as pl
from jax.experimental.pallas import tpu as pltpu
from jax.experimental.pallas import tpu_sc as plsc

D = 64
L = 50
LP = 56
NLANE = 16
NC = 2
NS = 16
NW = NC * NS
BATCH = 16384
CB = BATCH // NW


def _dist_sq(R, j, c):
    p0 = None
    p1 = None
    for q in range(4):
        x = R[j, pl.ds(NLANE * q, NLANE)]
        d = x - c[q]
        sq = d * d
        if q % 2 == 0:
            p0 = sq if p0 is None else p0 + sq
        else:
            p1 = sq if p1 is None else p1 + sq
    return jnp.sum(p0 + p1)


def _combine(R, b, wv, out_v):
    c = [R[L, pl.ds(NLANE * q, NLANE)] for q in range(4)]
    for j in range(L):
        wv[j] = _dist_sq(R, j, c)
    wacc = jnp.zeros((NLANE,), jnp.float32)
    for g in range(4):
        dv = wv[pl.ds(NLANE * g, NLANE)]
        valid = (lax.iota(jnp.int32, (NLANE,)) + NLANE * g) < L
        w = jnp.where(valid, jnp.exp(dv * -0.5), 0.0)
        wv[pl.ds(NLANE * g, NLANE)] = w
        wacc = wacc + w
    inv = 1.0 / (jnp.broadcast_to(jnp.sum(wacc) + 1e-8, (NLANE,)))
    acc = [jnp.zeros((NLANE,), jnp.float32) for _ in range(4)]
    for j in range(L):
        wj = wv[j]
        for q in range(4):
            acc[q] = acc[q] + wj * R[j, pl.ds(NLANE * q, NLANE)]
    for q in range(4):
        out_v[b, pl.ds(NLANE * q, NLANE)] = acc[q] * inv


@functools.partial(
    pl.kernel,
    out_type=jax.ShapeDtypeStruct((BATCH, D), jnp.float32),
    mesh=plsc.VectorSubcoreMesh(core_axis_name="c", subcore_axis_name="s"),
    scratch_types=[
        pltpu.VMEM((CB, LP), jnp.int32),
        pltpu.VMEM((CB, D), jnp.float32),
        pltpu.VMEM((LP, D), jnp.float32),
        pltpu.VMEM((LP, D), jnp.float32),
        pltpu.VMEM((D,), jnp.float32),
        pltpu.SemaphoreType.DMA,
        pltpu.SemaphoreType.DMA,
    ],
)
def _krembed(idx_hbm, table_hbm, out_hbm, idx_v, out_v, bufA, bufB, wv,
             semA, semB):
    wid = lax.axis_index("s") * NC + lax.axis_index("c")
    base = wid * CB
    pltpu.sync_copy(idx_hbm.at[pl.ds(base, CB)], idx_v)

    def fire(b, buf, sem):
        pltpu.async_copy(table_hbm.at[idx_v.at[b]], buf, sem)

    def drain(b, buf, sem):
        pltpu.make_async_copy(table_hbm.at[idx_v.at[b]], buf, sem).wait()

    fire(0, bufA, semA)

    def body(i, carry):
        b = i * 2
        fire(b + 1, bufB, semB)
        drain(b, bufA, semA)
        _combine(bufA, b, wv, out_v)

        @pl.when(b + 2 < CB)
        def _():
            fire(b + 2, bufA, semA)

        drain(b + 1, bufB, semB)
        _combine(bufB, b + 1, wv, out_v)
        return carry

    lax.fori_loop(0, CB // 2, body, 0)
    pltpu.sync_copy(out_v, out_hbm.at[pl.ds(base, CB)])


def kernel(context, center, embedding_weights):
    idx = jnp.concatenate(
        [context, jnp.broadcast_to(center[:, None], (BATCH, LP - L))], axis=1)
    return _krembed(idx, embedding_weights)


# speedup vs baseline: 2.3805x; 2.3805x over previous
import functools

import jax
import jax.numpy as jnp
from jax import lax
from jax.experimental import pallas as pl
from jax.experimental.pallas import tpu as pltpu
from jax.experimental.pallas import tpu_sc as plsc

D = 64
L = 50
LP = 56
NLANE = 16
NC = 2
NS = 16
NW = NC * NS
BATCH = 16384
CB = BATCH // NW


def _shuf(x, perm):
    return x.at[perm].get(mode="promise_in_bounds")


def _bitrev4(i):
    return ((i & 1) << 3) | ((i & 2) << 1) | ((i & 4) >> 1) | ((i & 8) >> 3)


def _partial_dist(R, j, c):
    s = None
    for q in range(4):
        x = R[j, pl.ds(NLANE * q, NLANE)]
        d = x - c[q]
        s = d * d if s is None else s + d * d
    return s


def _transpose_reduce(svecs, lane):
    vs = [svecs[_bitrev4(i)] for i in range(NLANE)]
    for r in (8, 4, 2, 1):
        perm = lane ^ r
        keep = (lane & r) == 0
        nxt = []
        for k in range(0, len(vs), 2):
            ra = vs[k] + _shuf(vs[k], perm)
            rb = vs[k + 1] + _shuf(vs[k + 1], perm)
            nxt.append(jnp.where(keep, ra, rb))
        vs = nxt
    return vs[0]


def _splat_sum(x, lane):
    for r in (8, 4, 2, 1):
        x = x + _shuf(x, lane ^ r)
    return x


def _combine(R, b, out_v):
    c = [R[L, pl.ds(NLANE * q, NLANE)] for q in range(4)]
    lane = lax.iota(jnp.int32, NLANE)
    zero = jnp.zeros((NLANE,), jnp.float32)
    wgs = []
    wacc = zero
    for g in range(4):
        n = min(L - g * NLANE, NLANE)
        svecs = [_partial_dist(R, g * NLANE + t, c) if t < n else zero
                 for t in range(NLANE)]
        dvec = _transpose_reduce(svecs, lane)
        w = jnp.exp(dvec * -0.5)
        if n < NLANE:
            w = jnp.where(lane < n, w, 0.0)
        wgs.append(w)
        wacc = wacc + w
    inv = 1.0 / (_splat_sum(wacc, lane) + 1e-8)
    acc = [zero] * 4
    for g in range(4):
        n = min(L - g * NLANE, NLANE)
        for t in range(n):
            j = g * NLANE + t
            wj = _shuf(wgs[g], lane * 0 + t)
            for q in range(4):
                acc[q] = acc[q] + wj * R[j, pl.ds(NLANE * q, NLANE)]
    for q in range(4):
        out_v[b, pl.ds(NLANE * q, NLANE)] = acc[q] * inv


@functools.partial(
    pl.kernel,
    out_type=jax.ShapeDtypeStruct((BATCH, D), jnp.float32),
    mesh=plsc.VectorSubcoreMesh(core_axis_name="c", subcore_axis_name="s"),
    compiler_params=pltpu.CompilerParams(use_tc_tiling_on_sc=False),
    scratch_types=[
        pltpu.VMEM((CB, LP), jnp.int32),
        pltpu.VMEM((CB, D), jnp.float32),
        pltpu.VMEM((LP, D), jnp.float32),
        pltpu.VMEM((LP, D), jnp.float32),
        pltpu.SemaphoreType.DMA,
        pltpu.SemaphoreType.DMA,
    ],
)
def _krembed(idx_hbm, table_hbm, out_hbm, idx_v, out_v, bufA, bufB,
             semA, semB):
    wid = lax.axis_index("s") * NC + lax.axis_index("c")
    base = wid * CB
    pltpu.sync_copy(idx_hbm.at[pl.ds(base, CB)], idx_v)

    def fire(b, buf, sem):
        pltpu.async_copy(table_hbm.at[idx_v.at[b]], buf, sem)

    def drain(b, buf, sem):
        pltpu.make_async_copy(table_hbm.at[idx_v.at[b]], buf, sem).wait()

    fire(0, bufA, semA)

    def body(i, carry):
        b = i * 2
        fire(b + 1, bufB, semB)
        drain(b, bufA, semA)
        _combine(bufA, b, out_v)

        @pl.when(b + 2 < CB)
        def _():
            fire(b + 2, bufA, semA)

        drain(b + 1, bufB, semB)
        _combine(bufB, b + 1, out_v)
        return carry

    lax.fori_loop(0, CB // 2, body, 0)
    pltpu.sync_copy(out_v, out_hbm.at[pl.ds(base, CB)])


def kernel(context, center, embedding_weights):
    idx = jnp.concatenate(
        [context, jnp.broadcast_to(center[:, None], (BATCH, LP - L))], axis=1)
    return _krembed(idx, embedding_weights)
